# baseline (speedup 1.0000x reference)
import functools

import jax
import jax.numpy as jnp
from jax import lax
from jax.experimental import pallas as pl
from jax.experimental.pallas import tpu as pltpu
from jax.experimental.pallas import tpu_sc as plsc

NUM_NODES = 100000
D = 512
B = 8192
K = 32
L = 16
NC = 2
NS = 16
NW = NC * NS
BPW = B // NW

_mesh = plsc.VectorSubcoreMesh(core_axis_name="c", subcore_axis_name="s")


@functools.partial(
    pl.kernel,
    out_type=(
        jax.ShapeDtypeStruct((B, D), jnp.float32),
        jax.ShapeDtypeStruct((B, D), jnp.float32),
        jax.ShapeDtypeStruct((B, D), jnp.float32),
    ),
    mesh=_mesh,
    scratch_types=[
        pltpu.VMEM((BPW,), jnp.int32),
        pltpu.VMEM((BPW, K), jnp.int32),
        pltpu.VMEM((BPW, K), jnp.int32),
        pltpu.VMEM((K, D), jnp.float32),
        pltpu.VMEM((1, D), jnp.float32),
        pltpu.SemaphoreType.DMA,
    ],
)
def _isne_sc(nidx_hbm, pidx_hbm, xidx_hbm, table_hbm,
             out_node, out_pos, out_neg,
             nidx_v, pidx_v, xidx_v, gbuf, row, sem):
    wid = lax.axis_index("s") * NC + lax.axis_index("c")
    base = wid * BPW

    pltpu.sync_copy(nidx_hbm.at[pl.ds(base, BPW)], nidx_v)
    pltpu.sync_copy(pidx_hbm.at[pl.ds(base, BPW)], pidx_v)
    pltpu.sync_copy(xidx_hbm.at[pl.ds(base, BPW)], xidx_v)

    def node_chunk(c, carry):
        pltpu.async_copy(table_hbm.at[nidx_v.at[pl.ds(c * K, K)]], gbuf, sem).wait()
        pltpu.sync_copy(gbuf, out_node.at[pl.ds(base + c * K, K)])
        return carry

    lax.fori_loop(0, BPW // K, node_chunk, 0)

    def run_side(idx_v, out_hbm):
        def per_anchor(a, carry):
            pltpu.async_copy(table_hbm.at[idx_v.at[a]], gbuf, sem).wait()

            def per_chunk(j, c2):
                acc = gbuf[0, pl.ds(j * L, L)]
                for k in range(1, K):
                    acc = acc + gbuf[k, pl.ds(j * L, L)]
                row[0, pl.ds(j * L, L)] = acc * (1.0 / K)
                return c2

            lax.fori_loop(0, D // L, per_chunk, 0)
            pltpu.sync_copy(row, out_hbm.at[pl.ds(base + a, 1)])
            return carry

        lax.fori_loop(0, BPW, per_anchor, 0)

    run_side(pidx_v, out_pos)
    run_side(xidx_v, out_neg)


def kernel(node_indices, pos_neighbor_indices, neg_neighbor_indices, node_parameters):
    return _isne_sc(
        node_indices.astype(jnp.int32),
        pos_neighbor_indices.astype(jnp.int32),
        neg_neighbor_indices.astype(jnp.int32),
        node_parameters,
    )


# speedup vs baseline: 2.8723x; 2.8723x over previous
import functools

import jax
import jax.numpy as jnp
from jax import lax
from jax.experimental import pallas as pl
from jax.experimental.pallas import tpu as pltpu
from jax.experimental.pallas import tpu_sc as plsc

NUM_NODES = 100000
D = 512
B = 8192
K = 32
L = 16
NC = 2
NS = 16
NW = NC * NS
BPW = B // NW
NBUF = 4
NCHUNK = BPW // NBUF
NODE_CHUNKS = BPW // K

_mesh = plsc.VectorSubcoreMesh(
    core_axis_name="c", subcore_axis_name="s", num_cores=NC, num_subcores=NS)


@functools.partial(
    pl.kernel,
    out_type=(
        jax.ShapeDtypeStruct((B, D), jnp.float32),
        jax.ShapeDtypeStruct((B, D), jnp.float32),
        jax.ShapeDtypeStruct((B, D), jnp.float32),
    ),
    mesh=_mesh,
    scratch_types=[
        pltpu.VMEM((BPW,), jnp.int32),
        pltpu.VMEM((BPW, K), jnp.int32),
        [pltpu.VMEM((K, D), jnp.float32) for _ in range(NBUF)],
        [pltpu.VMEM((1, D), jnp.float32) for _ in range(NBUF)],
        [pltpu.SemaphoreType.DMA for _ in range(NBUF)],
        [pltpu.SemaphoreType.DMA for _ in range(NBUF)],
    ],
)
def _isne_sc(nidx_hbm, pidx_hbm, xidx_hbm, table_hbm,
             out_node, out_pos, out_neg,
             nidx_v, sidx_v, gbufs, rows, gsems, osems):
    wid = lax.axis_index("s") * NC + lax.axis_index("c")
    wbase = wid * BPW

    pltpu.sync_copy(nidx_hbm.at[pl.ds(wbase, BPW)], nidx_v)

    inv_k = jnp.float32(1.0 / K)

    def run_side(idx_hbm, out_hbm):
        idx_v = sidx_v
        pltpu.sync_copy(idx_hbm.at[pl.ds(wbase, BPW)], idx_v)
        for b in range(NBUF):
            pltpu.async_copy(table_hbm.at[idx_v.at[b]], gbufs[b], gsems[b])

        def chunk_body(c, carry):
            for b in range(NBUF):
                a = c * NBUF + b
                pltpu.make_async_copy(
                    table_hbm.at[idx_v.at[a]], gbufs[b], gsems[b]).wait()

                @pl.when(c > 0)
                def _():
                    pltpu.make_async_copy(
                        rows[b], out_hbm.at[pl.ds(wbase + a, 1)],
                        osems[b]).wait()

                def reduce_chunk(j, c2):
                    sl = pl.ds(j * L, L)
                    acc = [gbufs[b][k, sl] for k in range(4)]
                    for k in range(4, K, 4):
                        for t in range(4):
                            acc[t] = acc[t] + gbufs[b][k + t, sl]
                    rows[b][0, sl] = ((acc[0] + acc[1]) + (acc[2] + acc[3])) * inv_k
                    return c2

                lax.fori_loop(0, D // L, reduce_chunk, 0, unroll=2)

                pltpu.async_copy(
                    rows[b], out_hbm.at[pl.ds(wbase + a, 1)], osems[b])

                @pl.when(c < NCHUNK - 1)
                def _():
                    pltpu.async_copy(
                        table_hbm.at[idx_v.at[a + NBUF]], gbufs[b], gsems[b])
            return carry

        lax.fori_loop(0, NCHUNK, chunk_body, 0)

        for b in range(NBUF):
            last = wbase + (NCHUNK - 1) * NBUF + b
            pltpu.make_async_copy(
                rows[b], out_hbm.at[pl.ds(last, 1)], osems[b]).wait()

    run_side(pidx_hbm, out_pos)
    run_side(xidx_hbm, out_neg)

    pltpu.async_copy(table_hbm.at[nidx_v.at[pl.ds(0, K)]], gbufs[0], gsems[0])
    pltpu.async_copy(table_hbm.at[nidx_v.at[pl.ds(K, K)]], gbufs[1], gsems[1])
    for c in range(NODE_CHUNKS):
        b = c % 2
        src = table_hbm.at[nidx_v.at[pl.ds(c * K, K)]]
        pltpu.make_async_copy(src, gbufs[b], gsems[b]).wait()
        dst = out_node.at[pl.ds(wbase + c * K, K)]
        if c + 2 < NODE_CHUNKS:
            pltpu.async_copy(gbufs[b], dst, osems[b])
            pltpu.make_async_copy(gbufs[b], dst, osems[b]).wait()
            pltpu.async_copy(
                table_hbm.at[nidx_v.at[pl.ds((c + 2) * K, K)]],
                gbufs[b], gsems[b])
        else:
            pltpu.sync_copy(gbufs[b], dst)


def kernel(node_indices, pos_neighbor_indices, neg_neighbor_indices, node_parameters):
    return _isne_sc(
        node_indices.astype(jnp.int32),
        pos_neighbor_indices.astype(jnp.int32),
        neg_neighbor_indices.astype(jnp.int32),
        node_parameters,
    )


# speedup vs baseline: 2.8815x; 1.0032x over previous
import functools

import jax
import jax.numpy as jnp
from jax import lax
from jax.experimental import pallas as pl
from jax.experimental.pallas import tpu as pltpu
from jax.experimental.pallas import tpu_sc as plsc

NUM_NODES = 100000
D = 512
B = 8192
K = 32
L = 16
NC = 2
NS = 16
NW = NC * NS
BPW = B // NW
NBUF = 4
NCHUNK = BPW // NBUF
NODE_CHUNKS = BPW // K

_mesh = plsc.VectorSubcoreMesh(
    core_axis_name="c", subcore_axis_name="s", num_cores=NC, num_subcores=NS)


@functools.partial(
    pl.kernel,
    out_type=(
        jax.ShapeDtypeStruct((B, D), jnp.float32),
        jax.ShapeDtypeStruct((B, D), jnp.float32),
        jax.ShapeDtypeStruct((B, D), jnp.float32),
    ),
    mesh=_mesh,
    scratch_types=[
        pltpu.VMEM((BPW,), jnp.int32),
        pltpu.VMEM((BPW, K), jnp.int32),
        [pltpu.VMEM((K, D), jnp.float32) for _ in range(NBUF)],
        [pltpu.VMEM((1, D), jnp.float32) for _ in range(NBUF)],
        [pltpu.SemaphoreType.DMA for _ in range(NBUF)],
        [pltpu.SemaphoreType.DMA for _ in range(NBUF)],
    ],
)
def _isne_sc(nidx_hbm, pidx_hbm, xidx_hbm, table_hbm,
             out_node, out_pos, out_neg,
             nidx_v, sidx_v, gbufs, rows, gsems, osems):
    wid = lax.axis_index("s") * NC + lax.axis_index("c")
    wbase = wid * BPW

    pltpu.sync_copy(nidx_hbm.at[pl.ds(wbase, BPW)], nidx_v)

    inv_k = jnp.float32(1.0 / K)

    def run_side(idx_hbm, out_hbm):
        idx_v = sidx_v
        pltpu.sync_copy(idx_hbm.at[pl.ds(wbase, BPW)], idx_v)
        for b in range(NBUF):
            pltpu.async_copy(table_hbm.at[idx_v.at[b]], gbufs[b], gsems[b])

        def chunk_body(c, carry):
            for b in range(NBUF):
                a = c * NBUF + b
                pltpu.make_async_copy(
                    table_hbm.at[idx_v.at[a]], gbufs[b], gsems[b]).wait()

                @pl.when(c > 0)
                def _():
                    pltpu.make_async_copy(
                        rows[b], out_hbm.at[pl.ds(wbase + a, 1)],
                        osems[b]).wait()

                def reduce_chunk(j, c2):
                    sl = pl.ds(j * L, L)
                    acc = [gbufs[b][k, sl] for k in range(4)]
                    for k in range(4, K, 4):
                        for t in range(4):
                            acc[t] = acc[t] + gbufs[b][k + t, sl]
                    rows[b][0, sl] = ((acc[0] + acc[1]) + (acc[2] + acc[3])) * inv_k
                    return c2

                lax.fori_loop(0, D // L, reduce_chunk, 0, unroll=4)

                pltpu.async_copy(
                    rows[b], out_hbm.at[pl.ds(wbase + a, 1)], osems[b])

                @pl.when(c < NCHUNK - 1)
                def _():
                    pltpu.async_copy(
                        table_hbm.at[idx_v.at[a + NBUF]], gbufs[b], gsems[b])
            return carry

        lax.fori_loop(0, NCHUNK, chunk_body, 0)

        for b in range(NBUF):
            last = wbase + (NCHUNK - 1) * NBUF + b
            pltpu.make_async_copy(
                rows[b], out_hbm.at[pl.ds(last, 1)], osems[b]).wait()

    run_side(pidx_hbm, out_pos)
    run_side(xidx_hbm, out_neg)

    pltpu.async_copy(table_hbm.at[nidx_v.at[pl.ds(0, K)]], gbufs[0], gsems[0])
    pltpu.async_copy(table_hbm.at[nidx_v.at[pl.ds(K, K)]], gbufs[1], gsems[1])
    for c in range(NODE_CHUNKS):
        b = c % 2
        src = table_hbm.at[nidx_v.at[pl.ds(c * K, K)]]
        pltpu.make_async_copy(src, gbufs[b], gsems[b]).wait()
        dst = out_node.at[pl.ds(wbase + c * K, K)]
        if c + 2 < NODE_CHUNKS:
            pltpu.async_copy(gbufs[b], dst, osems[b])
            pltpu.make_async_copy(gbufs[b], dst, osems[b]).wait()
            pltpu.async_copy(
                table_hbm.at[nidx_v.at[pl.ds((c + 2) * K, K)]],
                gbufs[b], gsems[b])
        else:
            pltpu.sync_copy(gbufs[b], dst)


def kernel(node_indices, pos_neighbor_indices, neg_neighbor_indices, node_parameters):
    return _isne_sc(
        node_indices.astype(jnp.int32),
        pos_neighbor_indices.astype(jnp.int32),
        neg_neighbor_indices.astype(jnp.int32),
        node_parameters,
    )
